# baseline (speedup 1.0000x reference)
import functools

import jax
import jax.numpy as jnp
from jax import lax
from jax.experimental import pallas as pl
from jax.experimental.pallas import tpu as pltpu
from jax.experimental.pallas import tpu_sc as plsc

N_NODES = 10000
D = 128
E = 320000

NC = 2
NS = 16
NW = NC * NS

CHUNK = 64
NBUF = 3
G0 = 87
G1 = 18
EDGES_PER_G = NBUF * CHUNK
E_PAD = NS * (G0 + G1) * EDGES_PER_G
N_ACC = 10240
ROWS_PER_TILE = N_ACC // NS
PAD_DST = N_NODES


def _sc_aggregate(x, row_h, col_h, zblock):
    mesh = plsc.VectorSubcoreMesh(core_axis_name="c", subcore_axis_name="s")

    @functools.partial(
        pl.kernel,
        mesh=mesh,
        out_type=jax.ShapeDtypeStruct((NC, N_ACC, D), jnp.float32),
        scratch_types=[
            pltpu.VMEM((CHUNK, D), jnp.float32),
            pltpu.VMEM((CHUNK, D), jnp.float32),
            pltpu.VMEM((CHUNK, D), jnp.float32),
            pltpu.VMEM((EDGES_PER_G,), jnp.int32),
            pltpu.VMEM((EDGES_PER_G,), jnp.int32),
            pltpu.VMEM((EDGES_PER_G,), jnp.int32),
            pltpu.VMEM((NBUF, CHUNK), jnp.int32),
            pltpu.VMEM((NBUF, CHUNK), jnp.int32),
            pltpu.VMEM((NBUF, CHUNK), jnp.int32),
            pltpu.VMEM_SHARED((N_ACC, D), jnp.float32),
            pltpu.SemaphoreType.DMA,
            pltpu.SemaphoreType.DMA,
            pltpu.SemaphoreType.DMA,
            pltpu.SemaphoreType.DMA,
            pltpu.SemaphoreType.DMA,
            pltpu.SemaphoreType.DMA,
        ],
    )
    def body(x_hbm, row_hbm, col_hbm, z_hbm, out_hbm,
             buf0, buf1, buf2, rr0, rr1, rr2, rc0, rc1, rc2, acc_sh,
             sem0, sem1, sem2, isem0, isem1, isem2):
        cid = lax.axis_index("c")
        sid = lax.axis_index("s")
        bufs = (buf0, buf1, buf2)
        sems = (sem0, sem1, sem2)
        rrow = (rr0, rr1, rr2)
        rcol = (rc0, rc1, rc2)
        isems = (isem0, isem1, isem2)

        pltpu.sync_copy(z_hbm, buf0)
        r0 = sid * ROWS_PER_TILE
        for b in range(ROWS_PER_TILE // CHUNK):
            pltpu.sync_copy(buf0, acc_sh.at[pl.ds(r0 + b * CHUNK, CHUNK)])
        plsc.subcore_barrier()

        def load_idx(copy, g_edge, k):
            copy(row_hbm.at[pl.ds(g_edge, EDGES_PER_G)], rrow[k])
            for b in range(NBUF):
                copy(col_hbm.at[pl.ds(g_edge + b * CHUNK, CHUNK)],
                     rcol[k].at[b])

        def wait_idx(k):
            pltpu.make_async_copy(row_hbm.at[pl.ds(0, EDGES_PER_G)],
                                  rrow[k], isems[k]).wait()
            for b in range(NBUF):
                pltpu.make_async_copy(col_hbm.at[pl.ds(0, CHUNK)],
                                      rcol[k].at[b], isems[k]).wait()

        def run(num_g, base_e):
            load_idx(pltpu.sync_copy, base_e, 0)
            load_idx(lambda s, d: pltpu.async_copy(s, d, isem1),
                     base_e + EDGES_PER_G, 1)
            for b in range(NBUF):
                pltpu.async_copy(x_hbm.at[rrow[0].at[pl.ds(b * CHUNK, CHUNK)]],
                                 bufs[b], sems[b])

            def triple(t, carry):
                for p in range(3):
                    g = 3 * t + p
                    pn = (p + 1) % 3
                    pf = (p + 2) % 3

                    @pl.when(g + 2 < num_g)
                    def _load_idx(pf=pf, g=g):
                        load_idx(
                            lambda s, d, pf=pf: pltpu.async_copy(s, d,
                                                                 isems[pf]),
                            base_e + (g + 2) * EDGES_PER_G, pf)

                    @pl.when(g + 1 < num_g)
                    def _wait_idx(pn=pn):
                        wait_idx(pn)

                    for b in range(NBUF):
                        pltpu.make_async_copy(x_hbm.at[pl.ds(0, CHUNK)],
                                              bufs[b], sems[b]).wait()
                        pltpu.sync_copy(bufs[b],
                                        acc_sh.at[rcol[p].at[b]],
                                        add=True)

                        @pl.when(g + 1 < num_g)
                        def _prefetch(pn=pn, b=b):
                            pltpu.async_copy(
                                x_hbm.at[rrow[pn].at[pl.ds(b * CHUNK, CHUNK)]],
                                bufs[b], sems[b])
                return carry

            lax.fori_loop(0, num_g // 3, triple, 0)

        @pl.when(cid == 0)
        def _fast_core():
            run(G0, sid * G0 * EDGES_PER_G)

        @pl.when(cid == 1)
        def _slow_core():
            run(G1, (NS * G0 + sid * G1) * EDGES_PER_G)

        plsc.subcore_barrier()

        pltpu.sync_copy(acc_sh.at[pl.ds(r0, ROWS_PER_TILE)],
                        out_hbm.at[cid, pl.ds(r0, ROWS_PER_TILE)])

    return body(x, row_h, col_h, zblock)


def _dense_r_body(x_ref, wr_ref, b_ref, o_ref):
    o_ref[...] = lax.dot_general(
        x_ref[...], wr_ref[...], (((1,), (1,)), ((), ())),
        preferred_element_type=jnp.float32) + b_ref[...]


def _dense_l_body(p0_ref, p1_ref, yr_ref, wl_ref, o_ref):
    agg = p0_ref[0] + p1_ref[0]
    o_ref[...] = lax.dot_general(
        agg, wl_ref[...], (((1,), (1,)), ((), ())),
        preferred_element_type=jnp.float32) + yr_ref[...]


def kernel(x, edge_index, W_l, b_l, W_r):
    npad = E_PAD - E
    row_h = jnp.concatenate([edge_index[0], jnp.zeros((npad,), jnp.int32)])
    col_h = jnp.concatenate([edge_index[1],
                             jnp.full((npad,), PAD_DST, jnp.int32)])
    zblock = jnp.zeros((CHUNK, D), jnp.float32)

    blk = 1000
    grid = (N_NODES // blk,)

    y_r = pl.pallas_call(
        _dense_r_body,
        grid=grid,
        in_specs=[
            pl.BlockSpec((blk, D), lambda i: (i, 0)),
            pl.BlockSpec((D, D), lambda i: (0, 0)),
            pl.BlockSpec((1, D), lambda i: (0, 0)),
        ],
        out_specs=pl.BlockSpec((blk, D), lambda i: (i, 0)),
        out_shape=jax.ShapeDtypeStruct((N_NODES, D), jnp.float32),
    )(x, W_r, b_l.reshape(1, D))

    p = _sc_aggregate(x, row_h, col_h, zblock)

    out = pl.pallas_call(
        _dense_l_body,
        grid=grid,
        in_specs=[
            pl.BlockSpec((1, blk, D), lambda i: (0, i, 0)),
            pl.BlockSpec((1, blk, D), lambda i: (1, i, 0)),
            pl.BlockSpec((blk, D), lambda i: (i, 0)),
            pl.BlockSpec((D, D), lambda i: (0, 0)),
        ],
        out_specs=pl.BlockSpec((blk, D), lambda i: (i, 0)),
        out_shape=jax.ShapeDtypeStruct((N_NODES, D), jnp.float32),
    )(p, p, y_r, W_l)
    return out


# speedup vs baseline: 1.1928x; 1.1928x over previous
import functools

import jax
import jax.numpy as jnp
from jax import lax
from jax.experimental import pallas as pl
from jax.experimental.pallas import tpu as pltpu
from jax.experimental.pallas import tpu_sc as plsc

N_NODES = 10000
D = 128
E = 320000

NC = 2
NS = 16

CHUNK = 128
N_CHUNKS = E // CHUNK
T0 = 2058
T1 = N_CHUNKS - T0
N_ACC = 10112
ROWS_PER_TILE = N_ACC // NS


def _sc_aggregate(x, ei):
    mesh = plsc.VectorSubcoreMesh(core_axis_name="c", subcore_axis_name="s")

    @functools.partial(
        pl.kernel,
        mesh=mesh,
        out_type=jax.ShapeDtypeStruct((NC, N_ACC, D), jnp.float32),
        scratch_types=[
            pltpu.VMEM((CHUNK, D), jnp.float32),
            pltpu.VMEM((CHUNK, D), jnp.float32),
            pltpu.VMEM((CHUNK, D), jnp.float32),
            pltpu.VMEM((2, CHUNK), jnp.int32),
            pltpu.VMEM((2, CHUNK), jnp.int32),
            pltpu.VMEM((2, CHUNK), jnp.int32),
            pltpu.VMEM_SHARED((N_ACC, D), jnp.float32),
            pltpu.SemaphoreType.DMA,
            pltpu.SemaphoreType.DMA,
            pltpu.SemaphoreType.DMA,
            pltpu.SemaphoreType.DMA,
            pltpu.SemaphoreType.DMA,
            pltpu.SemaphoreType.DMA,
        ],
    )
    def body(x_hbm, ei_hbm, z_hbm, out_hbm,
             buf0, buf1, buf2, ib0, ib1, ib2, acc_sh,
             sem0, sem1, sem2, isem0, isem1, isem2):
        cid = lax.axis_index("c")
        sid = lax.axis_index("s")
        bufs = (buf0, buf1, buf2)
        sems = (sem0, sem1, sem2)
        ibs = (ib0, ib1, ib2)
        isems = (isem0, isem1, isem2)

        pltpu.sync_copy(z_hbm, buf0)
        r0 = sid * ROWS_PER_TILE
        for b in range(ROWS_PER_TILE // CHUNK):
            pltpu.sync_copy(buf0, acc_sh.at[pl.ds(r0 + b * CHUNK, CHUNK)])
        rem = ROWS_PER_TILE % CHUNK
        if rem:
            nfull = ROWS_PER_TILE // CHUNK
            pltpu.sync_copy(buf0.at[pl.ds(0, rem)],
                            acc_sh.at[pl.ds(r0 + nfull * CHUNK, rem)])
        plsc.subcore_barrier()

        q0, rm0 = T0 // NS, T0 % NS
        q1, rm1 = T1 // NS, T1 % NS
        s32 = sid.astype(jnp.int32)
        start0 = s32 * q0 + jnp.minimum(s32, rm0)
        cnt0 = q0 + jnp.where(s32 < rm0, 1, 0)
        start1 = T0 + s32 * q1 + jnp.minimum(s32, rm1)
        cnt1 = q1 + jnp.where(s32 < rm1, 1, 0)
        start = jnp.where(cid == 0, start0, start1)
        cnt = jnp.where(cid == 0, cnt0, cnt1)

        def load_idx(copy, chunk_i, k):
            copy(ei_hbm.at[pl.ds(0, 2), pl.ds(chunk_i * CHUNK, CHUNK)],
                 ibs[k])

        def wait_idx(k):
            pltpu.make_async_copy(ei_hbm.at[pl.ds(0, 2), pl.ds(0, CHUNK)],
                                  ibs[k], isems[k]).wait()

        load_idx(pltpu.sync_copy, start, 0)
        load_idx(lambda s, d: pltpu.async_copy(s, d, isem1), start + 1, 1)
        load_idx(lambda s, d: pltpu.async_copy(s, d, isem2), start + 2, 2)
        pltpu.async_copy(x_hbm.at[ib0.at[0]], buf0, sem0)
        wait_idx(1)
        pltpu.async_copy(x_hbm.at[ib1.at[0]], buf1, sem1)

        def triple(t, carry):
            for p in range(3):
                i = 3 * t + p
                pn = (p + 2) % 3

                @pl.when(i < cnt)
                def _consume(p=p, i=i):
                    pltpu.make_async_copy(x_hbm.at[pl.ds(0, CHUNK)],
                                          bufs[p], sems[p]).wait()
                    pltpu.sync_copy(bufs[p], acc_sh.at[ibs[p].at[1]],
                                    add=True)

                @pl.when(i + 3 < cnt)
                def _load(p=p, i=i):
                    load_idx(lambda s, d, p=p: pltpu.async_copy(s, d,
                                                                isems[p]),
                             start + i + 3, p)

                @pl.when(i + 2 < cnt)
                def _gather(pn=pn, i=i):
                    wait_idx(pn)
                    pltpu.async_copy(x_hbm.at[ibs[pn].at[0]], bufs[pn],
                                     sems[pn])
            return carry

        lax.fori_loop(0, (cnt + 2) // 3, triple, 0)
        plsc.subcore_barrier()

        pltpu.sync_copy(acc_sh.at[pl.ds(r0, ROWS_PER_TILE)],
                        out_hbm.at[cid, pl.ds(r0, ROWS_PER_TILE)])

    zblock = jnp.zeros((CHUNK, D), jnp.float32)
    return body(x, ei, zblock)


def _dense_r_body(x_ref, wr_ref, b_ref, o_ref):
    o_ref[...] = lax.dot_general(
        x_ref[...], wr_ref[...], (((1,), (1,)), ((), ())),
        preferred_element_type=jnp.float32) + b_ref[...]


def _dense_l_body(p0_ref, p1_ref, yr_ref, wl_ref, o_ref):
    agg = p0_ref[0] + p1_ref[0]
    o_ref[...] = lax.dot_general(
        agg, wl_ref[...], (((1,), (1,)), ((), ())),
        preferred_element_type=jnp.float32) + yr_ref[...]


def kernel(x, edge_index, W_l, b_l, W_r):
    blk = 1000
    grid = (N_NODES // blk,)

    y_r = pl.pallas_call(
        _dense_r_body,
        grid=grid,
        in_specs=[
            pl.BlockSpec((blk, D), lambda i: (i, 0)),
            pl.BlockSpec((D, D), lambda i: (0, 0)),
            pl.BlockSpec((1, D), lambda i: (0, 0)),
        ],
        out_specs=pl.BlockSpec((blk, D), lambda i: (i, 0)),
        out_shape=jax.ShapeDtypeStruct((N_NODES, D), jnp.float32),
    )(x, W_r, b_l.reshape(1, D))

    p = _sc_aggregate(x, edge_index)

    out = pl.pallas_call(
        _dense_l_body,
        grid=grid,
        in_specs=[
            pl.BlockSpec((1, blk, D), lambda i: (0, i, 0)),
            pl.BlockSpec((1, blk, D), lambda i: (1, i, 0)),
            pl.BlockSpec((blk, D), lambda i: (i, 0)),
            pl.BlockSpec((D, D), lambda i: (0, 0)),
        ],
        out_specs=pl.BlockSpec((blk, D), lambda i: (i, 0)),
        out_shape=jax.ShapeDtypeStruct((N_NODES, D), jnp.float32),
    )(p, p, y_r, W_l)
    return out


# speedup vs baseline: 1.5408x; 1.2917x over previous
import functools

import jax
import jax.numpy as jnp
from jax import lax
from jax.experimental import pallas as pl
from jax.experimental.pallas import tpu as pltpu
from jax.experimental.pallas import tpu_sc as plsc

N_NODES = 10000
D = 128
E = 320000

NC = 2
NS = 16

CHUNK = 128
N_CHUNKS = E // CHUNK
T0 = 1433
T1 = N_CHUNKS - T0
N_ACC = 10112
ROWS_PER_TILE = N_ACC // NS


def _sc_aggregate(x, ei):
    mesh = plsc.VectorSubcoreMesh(core_axis_name="c", subcore_axis_name="s")

    @functools.partial(
        pl.kernel,
        mesh=mesh,
        out_type=jax.ShapeDtypeStruct((NC, N_ACC, D), jnp.float32),
        scratch_types=[
            pltpu.VMEM((CHUNK, D), jnp.float32),
            pltpu.VMEM((CHUNK, D), jnp.float32),
            pltpu.VMEM((CHUNK, D), jnp.float32),
            pltpu.VMEM((2, CHUNK), jnp.int32),
            pltpu.VMEM((2, CHUNK), jnp.int32),
            pltpu.VMEM((2, CHUNK), jnp.int32),
            pltpu.VMEM_SHARED((N_ACC, D), jnp.float32),
            pltpu.SemaphoreType.DMA,
            pltpu.SemaphoreType.DMA,
            pltpu.SemaphoreType.DMA,
            pltpu.SemaphoreType.DMA,
            pltpu.SemaphoreType.DMA,
            pltpu.SemaphoreType.DMA,
        ],
    )
    def body(x_hbm, ei_hbm, z_hbm, out_hbm,
             buf0, buf1, buf2, ib0, ib1, ib2, acc_sh,
             sem0, sem1, sem2, isem0, isem1, isem2):
        cid = lax.axis_index("c")
        sid = lax.axis_index("s")
        bufs = (buf0, buf1, buf2)
        sems = (sem0, sem1, sem2)
        ibs = (ib0, ib1, ib2)
        isems = (isem0, isem1, isem2)

        pltpu.sync_copy(z_hbm, buf0)
        r0 = sid * ROWS_PER_TILE
        for b in range(ROWS_PER_TILE // CHUNK):
            pltpu.sync_copy(buf0, acc_sh.at[pl.ds(r0 + b * CHUNK, CHUNK)])
        rem = ROWS_PER_TILE % CHUNK
        if rem:
            nfull = ROWS_PER_TILE // CHUNK
            pltpu.sync_copy(buf0.at[pl.ds(0, rem)],
                            acc_sh.at[pl.ds(r0 + nfull * CHUNK, rem)])
        plsc.subcore_barrier()

        q0, rm0 = T0 // NS, T0 % NS
        q1, rm1 = T1 // NS, T1 % NS
        s32 = sid.astype(jnp.int32)
        start0 = s32 * q0 + jnp.minimum(s32, rm0)
        cnt0 = q0 + jnp.where(s32 < rm0, 1, 0)
        start1 = T0 + s32 * q1 + jnp.minimum(s32, rm1)
        cnt1 = q1 + jnp.where(s32 < rm1, 1, 0)
        start = jnp.where(cid == 0, start0, start1)
        cnt = jnp.where(cid == 0, cnt0, cnt1)

        def load_idx(copy, chunk_i, k):
            copy(ei_hbm.at[pl.ds(0, 2), pl.ds(chunk_i * CHUNK, CHUNK)],
                 ibs[k])

        def wait_idx(k):
            pltpu.make_async_copy(ei_hbm.at[pl.ds(0, 2), pl.ds(0, CHUNK)],
                                  ibs[k], isems[k]).wait()

        load_idx(pltpu.sync_copy, start, 0)
        load_idx(lambda s, d: pltpu.async_copy(s, d, isem1), start + 1, 1)
        load_idx(lambda s, d: pltpu.async_copy(s, d, isem2), start + 2, 2)
        pltpu.async_copy(x_hbm.at[ib0.at[0]], buf0, sem0)
        wait_idx(1)
        pltpu.async_copy(x_hbm.at[ib1.at[0]], buf1, sem1)

        def triple(t, carry):
            for p in range(3):
                i = 3 * t + p
                pn = (p + 2) % 3

                @pl.when(i < cnt)
                def _consume(p=p, i=i):
                    pltpu.make_async_copy(x_hbm.at[pl.ds(0, CHUNK)],
                                          bufs[p], sems[p]).wait()
                    pltpu.sync_copy(bufs[p], acc_sh.at[ibs[p].at[1]],
                                    add=True)

                @pl.when(i + 3 < cnt)
                def _load(p=p, i=i):
                    load_idx(lambda s, d, p=p: pltpu.async_copy(s, d,
                                                                isems[p]),
                             start + i + 3, p)

                @pl.when(i + 2 < cnt)
                def _gather(pn=pn, i=i):
                    wait_idx(pn)
                    pltpu.async_copy(x_hbm.at[ibs[pn].at[0]], bufs[pn],
                                     sems[pn])
            return carry

        lax.fori_loop(0, (cnt + 2) // 3, triple, 0)
        plsc.subcore_barrier()

        pltpu.sync_copy(acc_sh.at[pl.ds(r0, ROWS_PER_TILE)],
                        out_hbm.at[cid, pl.ds(r0, ROWS_PER_TILE)])

    zblock = jnp.zeros((CHUNK, D), jnp.float32)
    return body(x, ei, zblock)


def _dense_r_body(x_ref, wr_ref, b_ref, o_ref):
    o_ref[...] = lax.dot_general(
        x_ref[...], wr_ref[...], (((1,), (1,)), ((), ())),
        preferred_element_type=jnp.float32) + b_ref[...]


def _dense_l_body(p0_ref, p1_ref, yr_ref, wl_ref, o_ref):
    agg = p0_ref[0] + p1_ref[0]
    o_ref[...] = lax.dot_general(
        agg, wl_ref[...], (((1,), (1,)), ((), ())),
        preferred_element_type=jnp.float32) + yr_ref[...]


def kernel(x, edge_index, W_l, b_l, W_r):
    blk = 1000
    grid = (N_NODES // blk,)

    y_r = pl.pallas_call(
        _dense_r_body,
        grid=grid,
        in_specs=[
            pl.BlockSpec((blk, D), lambda i: (i, 0)),
            pl.BlockSpec((D, D), lambda i: (0, 0)),
            pl.BlockSpec((1, D), lambda i: (0, 0)),
        ],
        out_specs=pl.BlockSpec((blk, D), lambda i: (i, 0)),
        out_shape=jax.ShapeDtypeStruct((N_NODES, D), jnp.float32),
    )(x, W_r, b_l.reshape(1, D))

    p = _sc_aggregate(x, edge_index)

    out = pl.pallas_call(
        _dense_l_body,
        grid=grid,
        in_specs=[
            pl.BlockSpec((1, blk, D), lambda i: (0, i, 0)),
            pl.BlockSpec((1, blk, D), lambda i: (1, i, 0)),
            pl.BlockSpec((blk, D), lambda i: (i, 0)),
            pl.BlockSpec((D, D), lambda i: (0, 0)),
        ],
        out_specs=pl.BlockSpec((blk, D), lambda i: (i, 0)),
        out_shape=jax.ShapeDtypeStruct((N_NODES, D), jnp.float32),
    )(p, p, y_r, W_l)
    return out
